# baseline (speedup 1.0000x reference)
import functools

import jax
import jax.numpy as jnp
from jax import lax
from jax.experimental import pallas as pl
from jax.experimental.pallas import tpu as pltpu
from jax.experimental.pallas import tpu_sc as plsc

NN = 10000
NE = 320000
HID = 64
NL = 3
NG = 64
DOUT = 64

NC = 2
NS = 16
NW = NC * NS
EPT = NE // NW
CHUNK = 80
NCHUNK = EPT // CHUNK
RPT = NN // NS
GRP = 5
NGRP = NCHUNK // GRP
GROWS = GRP * CHUNK

EBLK = 6400
NBLK = 2000


def _sc_mesh():
    return plsc.VectorSubcoreMesh(
        core_axis_name="c", subcore_axis_name="s", num_cores=NC, num_subcores=NS
    )


_SC_PARAMS = pltpu.CompilerParams(use_tc_tiling_on_sc=False,
                                 needs_layout_passes=False)


NPK = EPT // 2
NPK_IT = (NPK + 15) // 16


def _unpack_pairs(pk_v, emit):
    lanes = lax.iota(jnp.int32, 16)

    def body(it, carry):
        v = pk_v[pl.ds(it * 16, 16)]
        lo = jnp.bitwise_and(v, 0xFFFF)
        hi = lax.shift_right_logical(v, 16)
        p = 2 * (it * 16 + lanes)
        emit(p, lo)
        emit(p + 1, hi)
        return carry

    lax.fori_loop(0, NPK_IT, body, 0)


def _row_of(p):
    return lax.shift_right_logical(p * 52429, 22)


def _mk_emit(idx_v):
    def emit(p, x):
        r = _row_of(p)
        plsc.store_scatter(idx_v, [r, p - r * CHUNK], x)
    return emit


@functools.partial(
    pl.kernel,
    out_type=jax.ShapeDtypeStruct((NE, HID), jnp.bfloat16),
    mesh=_sc_mesh(),
    scratch_types=[
        pltpu.VMEM((NPK_IT * 16,), jnp.int32),
        pltpu.VMEM((NCHUNK + 1, CHUNK), jnp.int32),
        pltpu.VMEM((2, GROWS, HID), jnp.bfloat16),
        pltpu.SemaphoreType.DMA,
        pltpu.SemaphoreType.DMA,
    ],
    compiler_params=_SC_PARAMS,
)
def _sc_gather(table_hbm, pk_hbm, out_hbm, pk_v, idx_v, rows_v, gsem, osem):
    wid = lax.axis_index("s") * NC + lax.axis_index("c")
    pltpu.sync_copy(pk_hbm.at[wid], pk_v)
    _unpack_pairs(pk_v, _mk_emit(idx_v))
    base = wid * EPT
    gd = [None] * NGRP
    od = [None] * NGRP
    for g in range(NGRP):
        if g >= 2:
            od[g - 2].wait()
        buf = rows_v.at[g % 2]
        gd[g] = [
            pltpu.async_copy(
                table_hbm.at[idx_v.at[g * GRP + k]],
                buf.at[pl.ds(k * CHUNK, CHUNK)], gsem)
            for k in range(GRP)
        ]
        if g >= 1:
            for d in gd[g - 1]:
                d.wait()
            od[g - 1] = pltpu.async_copy(
                rows_v.at[(g - 1) % 2],
                out_hbm.at[pl.ds(base + (g - 1) * GROWS, GROWS)], osem)
    for d in gd[NGRP - 1]:
        d.wait()
    od[NGRP - 1] = pltpu.async_copy(
        rows_v.at[(NGRP - 1) % 2],
        out_hbm.at[pl.ds(base + (NGRP - 1) * GROWS, GROWS)], osem)
    od[NGRP - 2].wait()
    od[NGRP - 1].wait()


@functools.partial(
    pl.kernel,
    out_type=jax.ShapeDtypeStruct((NC * NN, HID), jnp.float32),
    mesh=_sc_mesh(),
    scratch_types=[
        pltpu.VMEM((NPK_IT * 16,), jnp.int32),
        pltpu.VMEM((NCHUNK + 1, CHUNK), jnp.int32),
        pltpu.VMEM((2, GROWS, HID), jnp.float32),
        pltpu.VMEM_SHARED((NN, HID), jnp.float32),
        pltpu.SemaphoreType.DMA,
        pltpu.SemaphoreType.DMA,
    ],
    compiler_params=_SC_PARAMS,
)
def _sc_scatter(m_hbm, pk_hbm, zeros_hbm, out_hbm, pk_v, idx_v, m_v, acc,
                lsem, ssem):
    cid = lax.axis_index("c")
    sid = lax.axis_index("s")
    wid = sid * NC + cid
    pltpu.sync_copy(zeros_hbm, m_v.at[0])
    pltpu.sync_copy(m_v.at[0], acc.at[pl.ds(sid * RPT, GROWS)])
    pltpu.sync_copy(m_v.at[0, pl.ds(0, RPT - GROWS)],
                    acc.at[pl.ds(sid * RPT + GROWS, RPT - GROWS)])
    pltpu.sync_copy(pk_hbm.at[wid], pk_v)
    _unpack_pairs(pk_v, _mk_emit(idx_v))
    plsc.subcore_barrier()
    base = wid * EPT

    ld = [None] * NGRP
    sd = [None] * NGRP

    def fire_scatters(g):
        buf = m_v.at[g % 2]
        sd[g] = [
            pltpu.async_copy(
                buf.at[pl.ds(k * CHUNK, CHUNK)],
                acc.at[idx_v.at[g * GRP + k]], ssem, add=True)
            for k in range(GRP)
        ]

    for g in range(NGRP):
        if g >= 2:
            for d in sd[g - 2]:
                d.wait()
        ld[g] = pltpu.async_copy(
            m_hbm.at[pl.ds(base + g * GROWS, GROWS)], m_v.at[g % 2], lsem)
        if g >= 1:
            ld[g - 1].wait()
            fire_scatters(g - 1)
    ld[NGRP - 1].wait()
    fire_scatters(NGRP - 1)
    for g in (NGRP - 2, NGRP - 1):
        for d in sd[g]:
            d.wait()
    plsc.subcore_barrier()
    obase = cid * NN + sid * RPT
    pltpu.sync_copy(acc.at[pl.ds(sid * RPT, GROWS)], m_v.at[0])
    pltpu.sync_copy(m_v.at[0], out_hbm.at[pl.ds(obase, GROWS)])
    pltpu.sync_copy(acc.at[pl.ds(sid * RPT + GROWS, RPT - GROWS)],
                    m_v.at[0, pl.ds(0, RPT - GROWS)])
    pltpu.sync_copy(m_v.at[0, pl.ds(0, RPT - GROWS)],
                    out_hbm.at[pl.ds(obase + GROWS, RPT - GROWS)])


@functools.partial(
    pl.kernel,
    out_type=jax.ShapeDtypeStruct((NC * NN, 16), jnp.float32),
    mesh=_sc_mesh(),
    scratch_types=[
        pltpu.VMEM((NCHUNK, CHUNK), jnp.int32),
        pltpu.VMEM((CHUNK, 16), jnp.float32),
        pltpu.VMEM((RPT, 16), jnp.float32),
        pltpu.VMEM_SHARED((NN, 16), jnp.float32),
        pltpu.SemaphoreType.DMA,
    ],
    compiler_params=_SC_PARAMS,
)
def _sc_counts(idx_hbm, ones_hbm, zeros_hbm, out_hbm, idx_v, ones_v, buf_v,
               acc, ssem):
    cid = lax.axis_index("c")
    sid = lax.axis_index("s")
    wid = sid * NC + cid
    pltpu.sync_copy(zeros_hbm, buf_v)
    pltpu.sync_copy(buf_v, acc.at[pl.ds(sid * RPT, RPT)])
    pltpu.sync_copy(idx_hbm.at[wid], idx_v)
    pltpu.sync_copy(ones_hbm, ones_v)
    plsc.subcore_barrier()

    ds = [
        pltpu.async_copy(ones_v, acc.at[idx_v.at[j]], ssem, add=True)
        for j in range(NCHUNK)
    ]
    for d in ds:
        d.wait()
    plsc.subcore_barrier()
    pltpu.sync_copy(acc.at[pl.ds(sid * RPT, RPT)], buf_v)
    pltpu.sync_copy(buf_v, out_hbm.at[pl.ds(cid * NN + sid * RPT, RPT)])


def _full(shape):
    return pl.BlockSpec(shape, lambda i: (0,) * len(shape))


def _embed_body(x_ref, w_ref, b_ref, w1h_ref, h_ref, hw_ref):
    h = jnp.dot(x_ref[...], w_ref[...], preferred_element_type=jnp.float32)
    h = h + b_ref[...]
    h_ref[...] = h
    hw_ref[...] = jnp.dot(
        h, w1h_ref[...], preferred_element_type=jnp.float32
    ).astype(jnp.bfloat16)


def _embed(x, w, b, w1h):
    grid = (NN // NBLK,)
    return pl.pallas_call(
        _embed_body,
        grid=grid,
        in_specs=[
            pl.BlockSpec((NBLK, 128), lambda i: (i, 0)),
            _full((128, HID)),
            _full((1, HID)),
            _full((HID, HID)),
        ],
        out_specs=[
            pl.BlockSpec((NBLK, HID), lambda i: (i, 0)),
            pl.BlockSpec((NBLK, HID), lambda i: (i, 0)),
        ],
        out_shape=[
            jax.ShapeDtypeStruct((NN, HID), jnp.float32),
            jax.ShapeDtypeStruct((NN, HID), jnp.bfloat16),
        ],
        compiler_params=pltpu.CompilerParams(
            dimension_semantics=("arbitrary",)
        ),
    )(x, w, b, w1h)


def _make_msg_body(first, want_eout):
    def body(*refs):
        if first:
            (eae_ref, eao_ref, g_ref, embw_ref, embb_ref, ew1_ref, eb1_ref,
             ew2_ref, eb2_ref, w1e_ref, b1_ref, w2_ref, b2_ref, *outs) = refs
        else:
            (e_ref, g_ref, ew1_ref, eb1_ref, ew2_ref, eb2_ref, w1e_ref,
             b1_ref, w2_ref, b2_ref, *outs) = refs
        if want_eout:
            m_ref, eout_ref, sums_ref = outs
        else:
            m_ref, sums_ref = outs

        if first:
            ea2 = jnp.concatenate([eae_ref[...], eao_ref[...]], axis=0)
            e = lax.dot_general(ea2, embw_ref[...], (((0,), (0,)), ((), ())),
                                preferred_element_type=jnp.float32)
            e = e + embb_ref[...]
        else:
            e = e_ref[...]
        eh = jnp.maximum(
            jnp.dot(e, ew1_ref[...], preferred_element_type=jnp.float32)
            + eb1_ref[...], 0.0)
        if want_eout:
            e2 = jnp.dot(eh, ew2_ref[...], preferred_element_type=jnp.float32)
            e2 = e2 + eb2_ref[...]
            eout_ref[...] = e2
        else:
            e2 = eh
        t = jnp.maximum(
            g_ref[...].astype(jnp.float32)
            + jnp.dot(e2, w1e_ref[...], preferred_element_type=jnp.float32)
            + b1_ref[...], 0.0)
        m = jnp.dot(t, w2_ref[...], preferred_element_type=jnp.float32)
        m = m + b2_ref[...]
        m_ref[...] = m

        @pl.when(pl.program_id(0) == 0)
        def _():
            sums_ref[...] = jnp.zeros((2, 2 * HID), jnp.float32)

        s1 = jnp.sum(m, axis=0, keepdims=True)
        s2 = jnp.sum(m * m, axis=0, keepdims=True)
        sums_ref[...] += jnp.concatenate([s1, s2], axis=0)

    return body


EBLK2 = EBLK // 2


def _msg(e_ins, g_p, embw, embb, ew1, eb1, ew2, eb2, w1e, b1, w2, b2,
         first, want_eout):
    grid = (NE // EBLK,)
    H2 = 2 * HID
    out_shape = [jax.ShapeDtypeStruct((NE // 2, H2), jnp.float32)]
    out_specs = [pl.BlockSpec((EBLK2, H2), lambda i: (i, 0))]
    if want_eout:
        out_shape.append(jax.ShapeDtypeStruct((NE // 2, H2), jnp.float32))
        out_specs.append(pl.BlockSpec((EBLK2, H2), lambda i: (i, 0)))
    out_shape.append(jax.ShapeDtypeStruct((2, H2), jnp.float32))
    out_specs.append(_full((2, H2)))
    nb2 = (NE // 2) // EBLK2
    if first:
        e_specs = [
            pl.BlockSpec((16, EBLK2), lambda i: (0, i)),
            pl.BlockSpec((16, EBLK2), lambda i: (0, nb2 + i)),
        ]
        emb_specs = [_full((32, H2)), _full((1, H2))]
        emb_args = (embw, embb)
    else:
        e_specs = [pl.BlockSpec((EBLK2, H2), lambda i: (i, 0))]
        emb_specs = []
        emb_args = ()
    wspecs = []
    for wmat in (ew1, eb1, ew2, eb2, w1e, b1, w2, b2):
        wspecs.append(_full(wmat.shape))
    return pl.pallas_call(
        _make_msg_body(first, want_eout),
        grid=grid,
        in_specs=e_specs + [
            pl.BlockSpec((EBLK2, H2), lambda i: (i, 0)),
        ] + emb_specs + wspecs,
        out_specs=out_specs,
        out_shape=out_shape,
        compiler_params=pltpu.CompilerParams(
            dimension_semantics=("arbitrary",)
        ),
    )(*e_ins, g_p, *emb_args, ew1, eb1, ew2, eb2, w1e, b1, w2, b2)


def _make_update_body(want_hw):
    def body(s0_ref, s1_ref, c0_ref, c1_ref, h_ref, sums_ref, bng_ref,
             bnb_ref, w1h_ref, h_out_ref, *rest):
        tot = sums_ref[:, :HID] + sums_ref[:, HID:]
        mu = tot[0:1, :] * (1.0 / NE)
        ms2 = tot[1:2, :] * (1.0 / NE)
        var = ms2 - mu * mu
        scale = bng_ref[...] * lax.rsqrt(var + 1e-5)
        shift = bnb_ref[...] - scale * mu
        cnt = c0_ref[:, 0:1] + c1_ref[:, 0:1]
        hn = scale * (s0_ref[...] + s1_ref[...]) + shift * cnt + h_ref[...]
        h_out_ref[...] = hn
        if want_hw:
            rest[0][...] = jnp.dot(
                hn, w1h_ref[...], preferred_element_type=jnp.float32
            ).astype(jnp.bfloat16)

    return body


def _update(s, counts, h, sums, bng, bnb, w1h, want_hw):
    grid = (NN // NBLK,)
    out_shape = [jax.ShapeDtypeStruct((NN, HID), jnp.float32)]
    out_specs = [pl.BlockSpec((NBLK, HID), lambda i: (i, 0))]
    if want_hw:
        out_shape.append(jax.ShapeDtypeStruct((NN, HID), jnp.bfloat16))
        out_specs.append(pl.BlockSpec((NBLK, HID), lambda i: (i, 0)))
    nb = NN // NBLK
    return pl.pallas_call(
        _make_update_body(want_hw),
        grid=grid,
        in_specs=[
            pl.BlockSpec((NBLK, HID), lambda i: (i, 0)),
            pl.BlockSpec((NBLK, HID), lambda i: (nb + i, 0)),
            pl.BlockSpec((NBLK, 16), lambda i: (i, 0)),
            pl.BlockSpec((NBLK, 16), lambda i: (nb + i, 0)),
            pl.BlockSpec((NBLK, HID), lambda i: (i, 0)),
            _full((2, 2 * HID)),
            _full((1, HID)),
            _full((1, HID)),
            _full((HID, HID)),
        ],
        out_specs=out_specs,
        out_shape=out_shape,
        compiler_params=pltpu.CompilerParams(
            dimension_semantics=("arbitrary",)
        ),
    )(s, s, counts, counts, h, sums, bng, bnb, w1h)


def _pool_body(h_ref, b_ref, rw1_ref, rb1_ref, rw2_ref, rb2_ref, z_ref,
               acc_ref):
    @pl.when(pl.program_id(0) == 0)
    def _():
        acc_ref[...] = jnp.zeros((NG, HID), jnp.float32)

    gid = lax.broadcasted_iota(jnp.int32, (NBLK, NG), 1)
    onehot = jnp.where(b_ref[...] == gid, 1.0, 0.0)
    acc_ref[...] += lax.dot_general(
        onehot, h_ref[...], (((0,), (0,)), ((), ())),
        preferred_element_type=jnp.float32)

    @pl.when(pl.program_id(0) == NN // NBLK - 1)
    def _():
        z = jnp.maximum(
            jnp.dot(acc_ref[...], rw1_ref[...],
                    preferred_element_type=jnp.float32) + rb1_ref[...], 0.0)
        z_ref[...] = jnp.dot(z, rw2_ref[...],
                             preferred_element_type=jnp.float32) + rb2_ref[...]


def _pool(h, batch2d, rw1, rb1, rw2, rb2):
    grid = (NN // NBLK,)
    return pl.pallas_call(
        _pool_body,
        grid=grid,
        in_specs=[
            pl.BlockSpec((NBLK, HID), lambda i: (i, 0)),
            pl.BlockSpec((NBLK, 1), lambda i: (i, 0)),
            _full((HID, HID)),
            _full((1, HID)),
            _full((HID, DOUT)),
            _full((1, DOUT)),
        ],
        out_specs=pl.BlockSpec((NG, DOUT), lambda i: (0, 0)),
        out_shape=jax.ShapeDtypeStruct((NG, DOUT), jnp.float32),
        scratch_shapes=[pltpu.VMEM((NG, HID), jnp.float32)],
        compiler_params=pltpu.CompilerParams(
            dimension_semantics=("arbitrary",)
        ),
    )(h, batch2d, rw1, rb1, rw2, rb2)


def kernel(x, edge_index, edge_attr, batch, node_emb_W, node_emb_b,
           edge_emb_W, edge_emb_b, W1, b1, W2, b2, bn_g, bn_b, eW1, eb1,
           eW2, eb2, rW1, rb1, rW2, rb2):
    f32 = jnp.float32
    pk = jnp.bitwise_or(
        edge_index[:, :NE // 2],
        jnp.left_shift(edge_index[:, NE // 2:], 16)).reshape(2, NW, NPK)
    pk = jnp.pad(pk, ((0, 0), (0, 0), (0, NPK_IT * 16 - NPK)))
    pk_src, pk_dst = pk[0], pk[1]
    dst_nat = edge_index[1].reshape(NW, NCHUNK, CHUNK)
    zeros64 = jnp.zeros((GROWS, HID), f32)
    zeros16 = jnp.zeros((RPT, 16), f32)
    ones16 = jnp.ones((CHUNK, 16), f32)

    row = lambda v: v.reshape(1, -1)

    counts = _sc_counts(dst_nat, ones16, zeros16)
    h, hw = _embed(x, node_emb_W, row(node_emb_b), W1[0, :HID])

    ea_t = edge_attr.T
    e_ins = (ea_t, ea_t)
    def bd(w):
        a, b = w.shape
        z = jnp.zeros((2 * a, 2 * b), f32)
        return z.at[:a, :b].set(w).at[a:, b:].set(w)

    def b2x(v):
        return jnp.concatenate([v, v]).reshape(1, -1)

    for l in range(NL):
        g = _sc_gather(hw, pk_src)
        g_p = g.reshape(NE // 2, 2 * HID)
        first = l == 0
        want_eout = l < NL - 1
        if want_eout:
            w1ed = bd(W1[l, HID:])
            b1d = b2x(b1[l])
        else:
            w1ed = bd(eW2[l] @ W1[l, HID:])
            b1d = b2x(b1[l] + eb2[l] @ W1[l, HID:])
        outs = _msg(e_ins, g_p, bd(edge_emb_W), b2x(edge_emb_b), bd(eW1[l]),
                    b2x(eb1[l]), bd(eW2[l]), b2x(eb2[l]), w1ed,
                    b1d, bd(W2[l]), b2x(b2[l]), first, want_eout)
        if want_eout:
            m_p, e_next, sums = outs
            e_ins = (e_next,)
        else:
            m_p, sums = outs
        s = _sc_scatter(m_p.reshape(NE, HID), pk_dst, zeros64)
        w1h_next = W1[l + 1, :HID] if want_eout else jnp.zeros((HID, HID), f32)
        ups = _update(s, counts, h, sums, bn_g[l].reshape(1, -1),
                      bn_b[l].reshape(1, -1), w1h_next, want_eout)
        if want_eout:
            h, hw = ups
        else:
            h = ups[0]

    return _pool(h, batch.reshape(NN, 1), rW1, row(rb1), rW2, row(rb2))


# speedup vs baseline: 1.5080x; 1.5080x over previous
import functools

import jax
import jax.numpy as jnp
from jax import lax
from jax.experimental import pallas as pl
from jax.experimental.pallas import tpu as pltpu
from jax.experimental.pallas import tpu_sc as plsc

NN = 10000
NE = 320000
HID = 64
NL = 3
NG = 64
DOUT = 64

NC = 2
NS = 16
NW = NC * NS
EPT = NE // NW
CHUNK = 80
NCHUNK = EPT // CHUNK
RPT = NN // NS
GRP = 5
NGRP = NCHUNK // GRP
GROWS = GRP * CHUNK

EBLK = 6400
NBLK = 2000


def _sc_mesh():
    return plsc.VectorSubcoreMesh(
        core_axis_name="c", subcore_axis_name="s", num_cores=NC, num_subcores=NS
    )


_SC_PARAMS = pltpu.CompilerParams(use_tc_tiling_on_sc=False,
                                 needs_layout_passes=False)


NPK = EPT // 2
NPK_IT = (NPK + 15) // 16


def _unpack_pairs(pk_v, emit):
    lanes = lax.iota(jnp.int32, 16)

    def body(it, carry):
        v = pk_v[pl.ds(it * 16, 16)]
        lo = jnp.bitwise_and(v, 0xFFFF)
        hi = lax.shift_right_logical(v, 16)
        p = 2 * (it * 16 + lanes)
        emit(p, lo)
        emit(p + 1, hi)
        return carry

    lax.fori_loop(0, NPK_IT, body, 0)


def _row_of(p):
    return lax.shift_right_logical(p * 52429, 22)


def _mk_emit(idx_v):
    def emit(p, x):
        r = _row_of(p)
        plsc.store_scatter(idx_v, [r, p - r * CHUNK], x)
    return emit


@functools.partial(
    pl.kernel,
    out_type=jax.ShapeDtypeStruct((NE, HID), jnp.float32),
    mesh=_sc_mesh(),
    scratch_types=[
        pltpu.VMEM((NPK_IT * 16,), jnp.int32),
        pltpu.VMEM((NCHUNK + 1, CHUNK), jnp.int32),
        pltpu.VMEM((2, GROWS, HID), jnp.float32),
        pltpu.SemaphoreType.DMA,
        pltpu.SemaphoreType.DMA,
    ],
    compiler_params=_SC_PARAMS,
)
def _sc_gather(table_hbm, pk_hbm, out_hbm, pk_v, idx_v, rows_v, gsem, osem):
    wid = lax.axis_index("s") * NC + lax.axis_index("c")
    pltpu.sync_copy(pk_hbm.at[wid], pk_v)
    _unpack_pairs(pk_v, _mk_emit(idx_v))
    base = wid * EPT
    gd = [None] * NGRP
    od = [None] * NGRP
    for g in range(NGRP):
        if g >= 2:
            od[g - 2].wait()
        buf = rows_v.at[g % 2]
        gd[g] = [
            pltpu.async_copy(
                table_hbm.at[idx_v.at[g * GRP + k]],
                buf.at[pl.ds(k * CHUNK, CHUNK)], gsem)
            for k in range(GRP)
        ]
        if g >= 1:
            for d in gd[g - 1]:
                d.wait()
            od[g - 1] = pltpu.async_copy(
                rows_v.at[(g - 1) % 2],
                out_hbm.at[pl.ds(base + (g - 1) * GROWS, GROWS)], osem)
    for d in gd[NGRP - 1]:
        d.wait()
    od[NGRP - 1] = pltpu.async_copy(
        rows_v.at[(NGRP - 1) % 2],
        out_hbm.at[pl.ds(base + (NGRP - 1) * GROWS, GROWS)], osem)
    od[NGRP - 2].wait()
    od[NGRP - 1].wait()


@functools.partial(
    pl.kernel,
    out_type=jax.ShapeDtypeStruct((NC * NN, HID), jnp.float32),
    mesh=_sc_mesh(),
    scratch_types=[
        pltpu.VMEM((NPK_IT * 16,), jnp.int32),
        pltpu.VMEM((NCHUNK + 1, CHUNK), jnp.int32),
        pltpu.VMEM((2, GROWS, HID), jnp.float32),
        pltpu.VMEM_SHARED((NN, HID), jnp.float32),
        pltpu.SemaphoreType.DMA,
        pltpu.SemaphoreType.DMA,
    ],
    compiler_params=_SC_PARAMS,
)
def _sc_scatter(m_hbm, pk_hbm, zeros_hbm, out_hbm, pk_v, idx_v, m_v, acc,
                lsem, ssem):
    cid = lax.axis_index("c")
    sid = lax.axis_index("s")
    wid = sid * NC + cid
    pltpu.sync_copy(zeros_hbm, m_v.at[0])
    pltpu.sync_copy(m_v.at[0], acc.at[pl.ds(sid * RPT, GROWS)])
    pltpu.sync_copy(m_v.at[0, pl.ds(0, RPT - GROWS)],
                    acc.at[pl.ds(sid * RPT + GROWS, RPT - GROWS)])
    pltpu.sync_copy(pk_hbm.at[wid], pk_v)
    _unpack_pairs(pk_v, _mk_emit(idx_v))
    plsc.subcore_barrier()
    base = wid * EPT

    ld = [None] * NGRP
    sd = [None] * NGRP

    def fire_scatters(g):
        buf = m_v.at[g % 2]
        sd[g] = [
            pltpu.async_copy(
                buf.at[pl.ds(k * CHUNK, CHUNK)],
                acc.at[idx_v.at[g * GRP + k]], ssem, add=True)
            for k in range(GRP)
        ]

    for g in range(NGRP):
        if g >= 2:
            for d in sd[g - 2]:
                d.wait()
        ld[g] = pltpu.async_copy(
            m_hbm.at[pl.ds(base + g * GROWS, GROWS)], m_v.at[g % 2], lsem)
        if g >= 1:
            ld[g - 1].wait()
            fire_scatters(g - 1)
    ld[NGRP - 1].wait()
    fire_scatters(NGRP - 1)
    for g in (NGRP - 2, NGRP - 1):
        for d in sd[g]:
            d.wait()
    plsc.subcore_barrier()
    obase = cid * NN + sid * RPT
    pltpu.sync_copy(acc.at[pl.ds(sid * RPT, GROWS)], m_v.at[0])
    pltpu.sync_copy(m_v.at[0], out_hbm.at[pl.ds(obase, GROWS)])
    pltpu.sync_copy(acc.at[pl.ds(sid * RPT + GROWS, RPT - GROWS)],
                    m_v.at[0, pl.ds(0, RPT - GROWS)])
    pltpu.sync_copy(m_v.at[0, pl.ds(0, RPT - GROWS)],
                    out_hbm.at[pl.ds(obase + GROWS, RPT - GROWS)])


@functools.partial(
    pl.kernel,
    out_type=jax.ShapeDtypeStruct((NC * NN, 16), jnp.float32),
    mesh=_sc_mesh(),
    scratch_types=[
        pltpu.VMEM((NCHUNK, CHUNK), jnp.int32),
        pltpu.VMEM((CHUNK, 16), jnp.float32),
        pltpu.VMEM((RPT, 16), jnp.float32),
        pltpu.VMEM_SHARED((NN, 16), jnp.float32),
        pltpu.SemaphoreType.DMA,
    ],
    compiler_params=_SC_PARAMS,
)
def _sc_counts(idx_hbm, ones_hbm, zeros_hbm, out_hbm, idx_v, ones_v, buf_v,
               acc, ssem):
    cid = lax.axis_index("c")
    sid = lax.axis_index("s")
    wid = sid * NC + cid
    pltpu.sync_copy(zeros_hbm, buf_v)
    pltpu.sync_copy(buf_v, acc.at[pl.ds(sid * RPT, RPT)])
    pltpu.sync_copy(idx_hbm.at[wid], idx_v)
    pltpu.sync_copy(ones_hbm, ones_v)
    plsc.subcore_barrier()

    ds = [
        pltpu.async_copy(ones_v, acc.at[idx_v.at[j]], ssem, add=True)
        for j in range(NCHUNK)
    ]
    for d in ds:
        d.wait()
    plsc.subcore_barrier()
    pltpu.sync_copy(acc.at[pl.ds(sid * RPT, RPT)], buf_v)
    pltpu.sync_copy(buf_v, out_hbm.at[pl.ds(cid * NN + sid * RPT, RPT)])


def _full(shape):
    return pl.BlockSpec(shape, lambda i: (0,) * len(shape))


def _embed_body(x_ref, w_ref, b_ref, w1h_ref, h_ref, hw_ref):
    h = jnp.dot(x_ref[...], w_ref[...], preferred_element_type=jnp.float32)
    h = h + b_ref[...]
    h_ref[...] = h
    hw_ref[...] = jnp.dot(h, w1h_ref[...], preferred_element_type=jnp.float32)


def _embed(x, w, b, w1h):
    grid = (NN // NBLK,)
    return pl.pallas_call(
        _embed_body,
        grid=grid,
        in_specs=[
            pl.BlockSpec((NBLK, 128), lambda i: (i, 0)),
            _full((128, HID)),
            _full((1, HID)),
            _full((HID, HID)),
        ],
        out_specs=[
            pl.BlockSpec((NBLK, HID), lambda i: (i, 0)),
            pl.BlockSpec((NBLK, HID), lambda i: (i, 0)),
        ],
        out_shape=[
            jax.ShapeDtypeStruct((NN, HID), jnp.float32),
            jax.ShapeDtypeStruct((NN, HID), jnp.float32),
        ],
        compiler_params=pltpu.CompilerParams(
            dimension_semantics=("arbitrary",)
        ),
    )(x, w, b, w1h)


def _make_msg_body(first, want_eout):
    def body(*refs):
        if first:
            (eae_ref, eao_ref, g_ref, embw_ref, embb_ref, ew1_ref, eb1_ref,
             ew2_ref, eb2_ref, w1e_ref, b1_ref, w2_ref, b2_ref, *outs) = refs
        else:
            (e_ref, g_ref, ew1_ref, eb1_ref, ew2_ref, eb2_ref, w1e_ref,
             b1_ref, w2_ref, b2_ref, *outs) = refs
        if want_eout:
            m_ref, eout_ref, sums_ref = outs
        else:
            m_ref, sums_ref = outs

        if first:
            ea2 = jnp.concatenate([eae_ref[...], eao_ref[...]], axis=0)
            e = lax.dot_general(ea2, embw_ref[...], (((0,), (0,)), ((), ())),
                                preferred_element_type=jnp.float32)
            e = e + embb_ref[...]
        else:
            e = e_ref[...]
        eh = jnp.maximum(
            jnp.dot(e, ew1_ref[...], preferred_element_type=jnp.float32)
            + eb1_ref[...], 0.0)
        if want_eout:
            e2 = jnp.dot(eh, ew2_ref[...], preferred_element_type=jnp.float32)
            e2 = e2 + eb2_ref[...]
            eout_ref[...] = e2
        else:
            e2 = eh
        t = jnp.maximum(
            g_ref[...]
            + jnp.dot(e2, w1e_ref[...], preferred_element_type=jnp.float32)
            + b1_ref[...], 0.0)
        m = jnp.dot(t, w2_ref[...], preferred_element_type=jnp.float32)
        m = m + b2_ref[...]
        m_ref[...] = m

        @pl.when(pl.program_id(0) == 0)
        def _():
            sums_ref[...] = jnp.zeros((2, 2 * HID), jnp.float32)

        s1 = jnp.sum(m, axis=0, keepdims=True)
        s2 = jnp.sum(m * m, axis=0, keepdims=True)
        sums_ref[...] += jnp.concatenate([s1, s2], axis=0)

    return body


EBLK2 = EBLK // 2


def _msg(e_ins, g_p, embw, embb, ew1, eb1, ew2, eb2, w1e, b1, w2, b2,
         first, want_eout):
    grid = (NE // EBLK,)
    H2 = 2 * HID
    out_shape = [jax.ShapeDtypeStruct((NE // 2, H2), jnp.float32)]
    out_specs = [pl.BlockSpec((EBLK2, H2), lambda i: (i, 0))]
    if want_eout:
        out_shape.append(jax.ShapeDtypeStruct((NE // 2, H2), jnp.float32))
        out_specs.append(pl.BlockSpec((EBLK2, H2), lambda i: (i, 0)))
    out_shape.append(jax.ShapeDtypeStruct((2, H2), jnp.float32))
    out_specs.append(_full((2, H2)))
    nb2 = (NE // 2) // EBLK2
    if first:
        e_specs = [
            pl.BlockSpec((16, EBLK2), lambda i: (0, i)),
            pl.BlockSpec((16, EBLK2), lambda i: (0, nb2 + i)),
        ]
        emb_specs = [_full((32, H2)), _full((1, H2))]
        emb_args = (embw, embb)
    else:
        e_specs = [pl.BlockSpec((EBLK2, H2), lambda i: (i, 0))]
        emb_specs = []
        emb_args = ()
    wspecs = []
    for wmat in (ew1, eb1, ew2, eb2, w1e, b1, w2, b2):
        wspecs.append(_full(wmat.shape))
    return pl.pallas_call(
        _make_msg_body(first, want_eout),
        grid=grid,
        in_specs=e_specs + [
            pl.BlockSpec((EBLK2, H2), lambda i: (i, 0)),
        ] + emb_specs + wspecs,
        out_specs=out_specs,
        out_shape=out_shape,
        compiler_params=pltpu.CompilerParams(
            dimension_semantics=("arbitrary",)
        ),
    )(*e_ins, g_p, *emb_args, ew1, eb1, ew2, eb2, w1e, b1, w2, b2)


def _make_update_body(want_hw):
    def body(s0_ref, s1_ref, c0_ref, c1_ref, h_ref, sums_ref, bng_ref,
             bnb_ref, w1h_ref, h_out_ref, *rest):
        tot = sums_ref[:, :HID] + sums_ref[:, HID:]
        mu = tot[0:1, :] * (1.0 / NE)
        ms2 = tot[1:2, :] * (1.0 / NE)
        var = ms2 - mu * mu
        scale = bng_ref[...] * lax.rsqrt(var + 1e-5)
        shift = bnb_ref[...] - scale * mu
        cnt = c0_ref[:, 0:1] + c1_ref[:, 0:1]
        hn = scale * (s0_ref[...] + s1_ref[...]) + shift * cnt + h_ref[...]
        h_out_ref[...] = hn
        if want_hw:
            rest[0][...] = jnp.dot(hn, w1h_ref[...],
                                   preferred_element_type=jnp.float32)

    return body


def _update(s, counts, h, sums, bng, bnb, w1h, want_hw):
    grid = (NN // NBLK,)
    out_shape = [jax.ShapeDtypeStruct((NN, HID), jnp.float32)]
    out_specs = [pl.BlockSpec((NBLK, HID), lambda i: (i, 0))]
    if want_hw:
        out_shape.append(jax.ShapeDtypeStruct((NN, HID), jnp.float32))
        out_specs.append(pl.BlockSpec((NBLK, HID), lambda i: (i, 0)))
    nb = NN // NBLK
    return pl.pallas_call(
        _make_update_body(want_hw),
        grid=grid,
        in_specs=[
            pl.BlockSpec((NBLK, HID), lambda i: (i, 0)),
            pl.BlockSpec((NBLK, HID), lambda i: (nb + i, 0)),
            pl.BlockSpec((NBLK, 16), lambda i: (i, 0)),
            pl.BlockSpec((NBLK, 16), lambda i: (nb + i, 0)),
            pl.BlockSpec((NBLK, HID), lambda i: (i, 0)),
            _full((2, 2 * HID)),
            _full((1, HID)),
            _full((1, HID)),
            _full((HID, HID)),
        ],
        out_specs=out_specs,
        out_shape=out_shape,
        compiler_params=pltpu.CompilerParams(
            dimension_semantics=("arbitrary",)
        ),
    )(s, s, counts, counts, h, sums, bng, bnb, w1h)


def _pool_body(h_ref, b_ref, rw1_ref, rb1_ref, rw2_ref, rb2_ref, z_ref,
               acc_ref):
    @pl.when(pl.program_id(0) == 0)
    def _():
        acc_ref[...] = jnp.zeros((NG, HID), jnp.float32)

    gid = lax.broadcasted_iota(jnp.int32, (NBLK, NG), 1)
    onehot = jnp.where(b_ref[...] == gid, 1.0, 0.0)
    acc_ref[...] += lax.dot_general(
        onehot, h_ref[...], (((0,), (0,)), ((), ())),
        preferred_element_type=jnp.float32)

    @pl.when(pl.program_id(0) == NN // NBLK - 1)
    def _():
        z = jnp.maximum(
            jnp.dot(acc_ref[...], rw1_ref[...],
                    preferred_element_type=jnp.float32) + rb1_ref[...], 0.0)
        z_ref[...] = jnp.dot(z, rw2_ref[...],
                             preferred_element_type=jnp.float32) + rb2_ref[...]


def _pool(h, batch2d, rw1, rb1, rw2, rb2):
    grid = (NN // NBLK,)
    return pl.pallas_call(
        _pool_body,
        grid=grid,
        in_specs=[
            pl.BlockSpec((NBLK, HID), lambda i: (i, 0)),
            pl.BlockSpec((NBLK, 1), lambda i: (i, 0)),
            _full((HID, HID)),
            _full((1, HID)),
            _full((HID, DOUT)),
            _full((1, DOUT)),
        ],
        out_specs=pl.BlockSpec((NG, DOUT), lambda i: (0, 0)),
        out_shape=jax.ShapeDtypeStruct((NG, DOUT), jnp.float32),
        scratch_shapes=[pltpu.VMEM((NG, HID), jnp.float32)],
        compiler_params=pltpu.CompilerParams(
            dimension_semantics=("arbitrary",)
        ),
    )(h, batch2d, rw1, rb1, rw2, rb2)


def kernel(x, edge_index, edge_attr, batch, node_emb_W, node_emb_b,
           edge_emb_W, edge_emb_b, W1, b1, W2, b2, bn_g, bn_b, eW1, eb1,
           eW2, eb2, rW1, rb1, rW2, rb2):
    f32 = jnp.float32
    pk = jnp.bitwise_or(
        edge_index[:, :NE // 2],
        jnp.left_shift(edge_index[:, NE // 2:], 16)).reshape(2, NW, NPK)
    pk = jnp.pad(pk, ((0, 0), (0, 0), (0, NPK_IT * 16 - NPK)))
    pk_src, pk_dst = pk[0], pk[1]
    dst_nat = edge_index[1].reshape(NW, NCHUNK, CHUNK)
    zeros64 = jnp.zeros((GROWS, HID), f32)
    zeros16 = jnp.zeros((RPT, 16), f32)
    ones16 = jnp.ones((CHUNK, 16), f32)

    row = lambda v: v.reshape(1, -1)

    counts = _sc_counts(dst_nat, ones16, zeros16)
    h, hw = _embed(x, node_emb_W, row(node_emb_b), W1[0, :HID])

    ea_t = edge_attr.T
    e_ins = (ea_t, ea_t)
    def bd(w):
        a, b = w.shape
        z = jnp.zeros((2 * a, 2 * b), f32)
        return z.at[:a, :b].set(w).at[a:, b:].set(w)

    def b2x(v):
        return jnp.concatenate([v, v]).reshape(1, -1)

    for l in range(NL):
        g = _sc_gather(hw, pk_src)
        g_p = g.reshape(NE // 2, 2 * HID)
        first = l == 0
        want_eout = l < NL - 1
        if want_eout:
            w1ed = bd(W1[l, HID:])
            b1d = b2x(b1[l])
        else:
            w1ed = bd(eW2[l] @ W1[l, HID:])
            b1d = b2x(b1[l] + eb2[l] @ W1[l, HID:])
        outs = _msg(e_ins, g_p, bd(edge_emb_W), b2x(edge_emb_b), bd(eW1[l]),
                    b2x(eb1[l]), bd(eW2[l]), b2x(eb2[l]), w1ed,
                    b1d, bd(W2[l]), b2x(b2[l]), first, want_eout)
        if want_eout:
            m_p, e_next, sums = outs
            e_ins = (e_next,)
        else:
            m_p, sums = outs
        s = _sc_scatter(m_p.reshape(NE, HID), pk_dst, zeros64)
        w1h_next = W1[l + 1, :HID] if want_eout else jnp.zeros((HID, HID), f32)
        ups = _update(s, counts, h, sums, bn_g[l].reshape(1, -1),
                      bn_b[l].reshape(1, -1), w1h_next, want_eout)
        if want_eout:
            h, hw = ups
        else:
            h = ups[0]

    return _pool(h, batch.reshape(NN, 1), rW1, row(rb1), rW2, row(rb2))


# speedup vs baseline: 1.6000x; 1.0610x over previous
import functools

import jax
import jax.numpy as jnp
from jax import lax
from jax.experimental import pallas as pl
from jax.experimental.pallas import tpu as pltpu
from jax.experimental.pallas import tpu_sc as plsc

NN = 10000
NE = 320000
HID = 64
NL = 3
NG = 64
DOUT = 64

NC = 2
NS = 16
NW = NC * NS
EPT = NE // NW
CHUNK = 80
NCHUNK = EPT // CHUNK
RPT = NN // NS
GRP = 5
NGRP = NCHUNK // GRP
GROWS = GRP * CHUNK

EBLK = 12800
NBLK = 2000


def _sc_mesh():
    return plsc.VectorSubcoreMesh(
        core_axis_name="c", subcore_axis_name="s", num_cores=NC, num_subcores=NS
    )


_SC_PARAMS = pltpu.CompilerParams(use_tc_tiling_on_sc=False,
                                 needs_layout_passes=False)


NPK = EPT // 2
NPK_IT = (NPK + 15) // 16


def _unpack_pairs(pk_v, emit):
    lanes = lax.iota(jnp.int32, 16)

    def body(it, carry):
        v = pk_v[pl.ds(it * 16, 16)]
        lo = jnp.bitwise_and(v, 0xFFFF)
        hi = lax.shift_right_logical(v, 16)
        p = 2 * (it * 16 + lanes)
        emit(p, lo)
        emit(p + 1, hi)
        return carry

    lax.fori_loop(0, NPK_IT, body, 0)


def _row_of(p):
    return lax.shift_right_logical(p * 52429, 22)


def _mk_emit(idx_v):
    def emit(p, x):
        r = _row_of(p)
        plsc.store_scatter(idx_v, [r, p - r * CHUNK], x)
    return emit


@functools.partial(
    pl.kernel,
    out_type=jax.ShapeDtypeStruct((NE, HID), jnp.float32),
    mesh=_sc_mesh(),
    scratch_types=[
        pltpu.VMEM((NPK_IT * 16,), jnp.int32),
        pltpu.VMEM((NCHUNK + 1, CHUNK), jnp.int32),
        pltpu.VMEM((2, GROWS, HID), jnp.float32),
        pltpu.SemaphoreType.DMA,
        pltpu.SemaphoreType.DMA,
    ],
    compiler_params=_SC_PARAMS,
)
def _sc_gather(table_hbm, pk_hbm, out_hbm, pk_v, idx_v, rows_v, gsem, osem):
    wid = lax.axis_index("s") * NC + lax.axis_index("c")
    pltpu.sync_copy(pk_hbm.at[wid], pk_v)
    _unpack_pairs(pk_v, _mk_emit(idx_v))
    base = wid * EPT
    gd = [None] * NGRP
    od = [None] * NGRP
    for g in range(NGRP):
        if g >= 2:
            od[g - 2].wait()
        buf = rows_v.at[g % 2]
        gd[g] = [
            pltpu.async_copy(
                table_hbm.at[idx_v.at[g * GRP + k]],
                buf.at[pl.ds(k * CHUNK, CHUNK)], gsem)
            for k in range(GRP)
        ]
        if g >= 1:
            for d in gd[g - 1]:
                d.wait()
            od[g - 1] = pltpu.async_copy(
                rows_v.at[(g - 1) % 2],
                out_hbm.at[pl.ds(base + (g - 1) * GROWS, GROWS)], osem)
    for d in gd[NGRP - 1]:
        d.wait()
    od[NGRP - 1] = pltpu.async_copy(
        rows_v.at[(NGRP - 1) % 2],
        out_hbm.at[pl.ds(base + (NGRP - 1) * GROWS, GROWS)], osem)
    od[NGRP - 2].wait()
    od[NGRP - 1].wait()


@functools.partial(
    pl.kernel,
    out_type=jax.ShapeDtypeStruct((NC * NN, HID), jnp.float32),
    mesh=_sc_mesh(),
    scratch_types=[
        pltpu.VMEM((NPK_IT * 16,), jnp.int32),
        pltpu.VMEM((NCHUNK + 1, CHUNK), jnp.int32),
        pltpu.VMEM((2, GROWS, HID), jnp.float32),
        pltpu.VMEM_SHARED((NN, HID), jnp.float32),
        pltpu.SemaphoreType.DMA,
        pltpu.SemaphoreType.DMA,
    ],
    compiler_params=_SC_PARAMS,
)
def _sc_scatter(m_hbm, pk_hbm, zeros_hbm, out_hbm, pk_v, idx_v, m_v, acc,
                lsem, ssem):
    cid = lax.axis_index("c")
    sid = lax.axis_index("s")
    wid = sid * NC + cid
    pltpu.sync_copy(zeros_hbm, m_v.at[0])
    pltpu.sync_copy(m_v.at[0], acc.at[pl.ds(sid * RPT, GROWS)])
    pltpu.sync_copy(m_v.at[0, pl.ds(0, RPT - GROWS)],
                    acc.at[pl.ds(sid * RPT + GROWS, RPT - GROWS)])
    pltpu.sync_copy(pk_hbm.at[wid], pk_v)
    _unpack_pairs(pk_v, _mk_emit(idx_v))
    plsc.subcore_barrier()
    base = wid * EPT

    ld = [None] * NGRP
    sd = [None] * NGRP

    def fire_scatters(g):
        buf = m_v.at[g % 2]
        sd[g] = [
            pltpu.async_copy(
                buf.at[pl.ds(k * CHUNK, CHUNK)],
                acc.at[idx_v.at[g * GRP + k]], ssem, add=True)
            for k in range(GRP)
        ]

    for g in range(NGRP):
        if g >= 2:
            for d in sd[g - 2]:
                d.wait()
        ld[g] = pltpu.async_copy(
            m_hbm.at[pl.ds(base + g * GROWS, GROWS)], m_v.at[g % 2], lsem)
        if g >= 1:
            ld[g - 1].wait()
            fire_scatters(g - 1)
    ld[NGRP - 1].wait()
    fire_scatters(NGRP - 1)
    for g in (NGRP - 2, NGRP - 1):
        for d in sd[g]:
            d.wait()
    plsc.subcore_barrier()
    obase = cid * NN + sid * RPT
    pltpu.sync_copy(acc.at[pl.ds(sid * RPT, GROWS)], m_v.at[0])
    pltpu.sync_copy(m_v.at[0], out_hbm.at[pl.ds(obase, GROWS)])
    pltpu.sync_copy(acc.at[pl.ds(sid * RPT + GROWS, RPT - GROWS)],
                    m_v.at[0, pl.ds(0, RPT - GROWS)])
    pltpu.sync_copy(m_v.at[0, pl.ds(0, RPT - GROWS)],
                    out_hbm.at[pl.ds(obase + GROWS, RPT - GROWS)])


@functools.partial(
    pl.kernel,
    out_type=jax.ShapeDtypeStruct((NC * NN, 16), jnp.float32),
    mesh=_sc_mesh(),
    scratch_types=[
        pltpu.VMEM((NCHUNK, CHUNK), jnp.int32),
        pltpu.VMEM((CHUNK, 16), jnp.float32),
        pltpu.VMEM((RPT, 16), jnp.float32),
        pltpu.VMEM_SHARED((NN, 16), jnp.float32),
        pltpu.SemaphoreType.DMA,
    ],
    compiler_params=_SC_PARAMS,
)
def _sc_counts(idx_hbm, ones_hbm, zeros_hbm, out_hbm, idx_v, ones_v, buf_v,
               acc, ssem):
    cid = lax.axis_index("c")
    sid = lax.axis_index("s")
    wid = sid * NC + cid
    pltpu.sync_copy(zeros_hbm, buf_v)
    pltpu.sync_copy(buf_v, acc.at[pl.ds(sid * RPT, RPT)])
    pltpu.sync_copy(idx_hbm.at[wid], idx_v)
    pltpu.sync_copy(ones_hbm, ones_v)
    plsc.subcore_barrier()

    ds = [
        pltpu.async_copy(ones_v, acc.at[idx_v.at[j]], ssem, add=True)
        for j in range(NCHUNK)
    ]
    for d in ds:
        d.wait()
    plsc.subcore_barrier()
    pltpu.sync_copy(acc.at[pl.ds(sid * RPT, RPT)], buf_v)
    pltpu.sync_copy(buf_v, out_hbm.at[pl.ds(cid * NN + sid * RPT, RPT)])


def _full(shape):
    return pl.BlockSpec(shape, lambda i: (0,) * len(shape))


def _embed_body(x_ref, w_ref, b_ref, w1h_ref, h_ref, hw_ref):
    h = jnp.dot(x_ref[...], w_ref[...], preferred_element_type=jnp.float32)
    h = h + b_ref[...]
    h_ref[...] = h
    hw_ref[...] = jnp.dot(h, w1h_ref[...], preferred_element_type=jnp.float32)


def _embed(x, w, b, w1h):
    grid = (NN // NBLK,)
    return pl.pallas_call(
        _embed_body,
        grid=grid,
        in_specs=[
            pl.BlockSpec((NBLK, 128), lambda i: (i, 0)),
            _full((128, HID)),
            _full((1, HID)),
            _full((HID, HID)),
        ],
        out_specs=[
            pl.BlockSpec((NBLK, HID), lambda i: (i, 0)),
            pl.BlockSpec((NBLK, HID), lambda i: (i, 0)),
        ],
        out_shape=[
            jax.ShapeDtypeStruct((NN, HID), jnp.float32),
            jax.ShapeDtypeStruct((NN, HID), jnp.float32),
        ],
        compiler_params=pltpu.CompilerParams(
            dimension_semantics=("arbitrary",)
        ),
    )(x, w, b, w1h)


def _make_msg_body(first, want_eout):
    def body(*refs):
        if first:
            (eae_ref, eao_ref, g_ref, embw_ref, embb_ref, ew1_ref, eb1_ref,
             ew2_ref, eb2_ref, w1e_ref, b1_ref, w2_ref, b2_ref, *outs) = refs
        else:
            (e_ref, g_ref, ew1_ref, eb1_ref, ew2_ref, eb2_ref, w1e_ref,
             b1_ref, w2_ref, b2_ref, *outs) = refs
        if want_eout:
            m_ref, eout_ref, sums_ref = outs
        else:
            m_ref, sums_ref = outs

        if first:
            ea2 = jnp.concatenate([eae_ref[...], eao_ref[...]], axis=0)
            e = lax.dot_general(ea2, embw_ref[...], (((0,), (0,)), ((), ())),
                                preferred_element_type=jnp.float32)
            e = e + embb_ref[...]
        else:
            e = e_ref[...]
        eh = jnp.maximum(
            jnp.dot(e, ew1_ref[...], preferred_element_type=jnp.float32)
            + eb1_ref[...], 0.0)
        if want_eout:
            e2 = jnp.dot(eh, ew2_ref[...], preferred_element_type=jnp.float32)
            e2 = e2 + eb2_ref[...]
            eout_ref[...] = e2
        else:
            e2 = eh
        t = jnp.maximum(
            g_ref[...]
            + jnp.dot(e2, w1e_ref[...], preferred_element_type=jnp.float32)
            + b1_ref[...], 0.0)
        m = jnp.dot(t, w2_ref[...], preferred_element_type=jnp.float32)
        m = m + b2_ref[...]
        m_ref[...] = m

        @pl.when(pl.program_id(0) == 0)
        def _():
            sums_ref[...] = jnp.zeros((2, 2 * HID), jnp.float32)

        s1 = jnp.sum(m, axis=0, keepdims=True)
        s2 = jnp.sum(m * m, axis=0, keepdims=True)
        sums_ref[...] += jnp.concatenate([s1, s2], axis=0)

    return body


EBLK2 = EBLK // 2


def _msg(e_ins, g_p, embw, embb, ew1, eb1, ew2, eb2, w1e, b1, w2, b2,
         first, want_eout):
    grid = (NE // EBLK,)
    H2 = 2 * HID
    out_shape = [jax.ShapeDtypeStruct((NE // 2, H2), jnp.float32)]
    out_specs = [pl.BlockSpec((EBLK2, H2), lambda i: (i, 0))]
    if want_eout:
        out_shape.append(jax.ShapeDtypeStruct((NE // 2, H2), jnp.float32))
        out_specs.append(pl.BlockSpec((EBLK2, H2), lambda i: (i, 0)))
    out_shape.append(jax.ShapeDtypeStruct((2, H2), jnp.float32))
    out_specs.append(_full((2, H2)))
    nb2 = (NE // 2) // EBLK2
    if first:
        e_specs = [
            pl.BlockSpec((16, EBLK2), lambda i: (0, i)),
            pl.BlockSpec((16, EBLK2), lambda i: (0, nb2 + i)),
        ]
        emb_specs = [_full((32, H2)), _full((1, H2))]
        emb_args = (embw, embb)
    else:
        e_specs = [pl.BlockSpec((EBLK2, H2), lambda i: (i, 0))]
        emb_specs = []
        emb_args = ()
    wspecs = []
    for wmat in (ew1, eb1, ew2, eb2, w1e, b1, w2, b2):
        wspecs.append(_full(wmat.shape))
    return pl.pallas_call(
        _make_msg_body(first, want_eout),
        grid=grid,
        in_specs=e_specs + [
            pl.BlockSpec((EBLK2, H2), lambda i: (i, 0)),
        ] + emb_specs + wspecs,
        out_specs=out_specs,
        out_shape=out_shape,
        compiler_params=pltpu.CompilerParams(
            dimension_semantics=("arbitrary",)
        ),
    )(*e_ins, g_p, *emb_args, ew1, eb1, ew2, eb2, w1e, b1, w2, b2)


def _make_update_body(want_hw):
    def body(s0_ref, s1_ref, c0_ref, c1_ref, h_ref, sums_ref, bng_ref,
             bnb_ref, w1h_ref, h_out_ref, *rest):
        tot = sums_ref[:, :HID] + sums_ref[:, HID:]
        mu = tot[0:1, :] * (1.0 / NE)
        ms2 = tot[1:2, :] * (1.0 / NE)
        var = ms2 - mu * mu
        scale = bng_ref[...] * lax.rsqrt(var + 1e-5)
        shift = bnb_ref[...] - scale * mu
        cnt = c0_ref[:, 0:1] + c1_ref[:, 0:1]
        hn = scale * (s0_ref[...] + s1_ref[...]) + shift * cnt + h_ref[...]
        h_out_ref[...] = hn
        if want_hw:
            rest[0][...] = jnp.dot(hn, w1h_ref[...],
                                   preferred_element_type=jnp.float32)

    return body


def _update(s, counts, h, sums, bng, bnb, w1h, want_hw):
    grid = (NN // NBLK,)
    out_shape = [jax.ShapeDtypeStruct((NN, HID), jnp.float32)]
    out_specs = [pl.BlockSpec((NBLK, HID), lambda i: (i, 0))]
    if want_hw:
        out_shape.append(jax.ShapeDtypeStruct((NN, HID), jnp.float32))
        out_specs.append(pl.BlockSpec((NBLK, HID), lambda i: (i, 0)))
    nb = NN // NBLK
    return pl.pallas_call(
        _make_update_body(want_hw),
        grid=grid,
        in_specs=[
            pl.BlockSpec((NBLK, HID), lambda i: (i, 0)),
            pl.BlockSpec((NBLK, HID), lambda i: (nb + i, 0)),
            pl.BlockSpec((NBLK, 16), lambda i: (i, 0)),
            pl.BlockSpec((NBLK, 16), lambda i: (nb + i, 0)),
            pl.BlockSpec((NBLK, HID), lambda i: (i, 0)),
            _full((2, 2 * HID)),
            _full((1, HID)),
            _full((1, HID)),
            _full((HID, HID)),
        ],
        out_specs=out_specs,
        out_shape=out_shape,
        compiler_params=pltpu.CompilerParams(
            dimension_semantics=("arbitrary",)
        ),
    )(s, s, counts, counts, h, sums, bng, bnb, w1h)


def _pool_body(h_ref, b_ref, rw1_ref, rb1_ref, rw2_ref, rb2_ref, z_ref,
               acc_ref):
    @pl.when(pl.program_id(0) == 0)
    def _():
        acc_ref[...] = jnp.zeros((NG, HID), jnp.float32)

    gid = lax.broadcasted_iota(jnp.int32, (NBLK, NG), 1)
    onehot = jnp.where(b_ref[...] == gid, 1.0, 0.0)
    acc_ref[...] += lax.dot_general(
        onehot, h_ref[...], (((0,), (0,)), ((), ())),
        preferred_element_type=jnp.float32)

    @pl.when(pl.program_id(0) == NN // NBLK - 1)
    def _():
        z = jnp.maximum(
            jnp.dot(acc_ref[...], rw1_ref[...],
                    preferred_element_type=jnp.float32) + rb1_ref[...], 0.0)
        z_ref[...] = jnp.dot(z, rw2_ref[...],
                             preferred_element_type=jnp.float32) + rb2_ref[...]


def _pool(h, batch2d, rw1, rb1, rw2, rb2):
    grid = (NN // NBLK,)
    return pl.pallas_call(
        _pool_body,
        grid=grid,
        in_specs=[
            pl.BlockSpec((NBLK, HID), lambda i: (i, 0)),
            pl.BlockSpec((NBLK, 1), lambda i: (i, 0)),
            _full((HID, HID)),
            _full((1, HID)),
            _full((HID, DOUT)),
            _full((1, DOUT)),
        ],
        out_specs=pl.BlockSpec((NG, DOUT), lambda i: (0, 0)),
        out_shape=jax.ShapeDtypeStruct((NG, DOUT), jnp.float32),
        scratch_shapes=[pltpu.VMEM((NG, HID), jnp.float32)],
        compiler_params=pltpu.CompilerParams(
            dimension_semantics=("arbitrary",)
        ),
    )(h, batch2d, rw1, rb1, rw2, rb2)


def kernel(x, edge_index, edge_attr, batch, node_emb_W, node_emb_b,
           edge_emb_W, edge_emb_b, W1, b1, W2, b2, bn_g, bn_b, eW1, eb1,
           eW2, eb2, rW1, rb1, rW2, rb2):
    f32 = jnp.float32
    pk = jnp.bitwise_or(
        edge_index[:, :NE // 2],
        jnp.left_shift(edge_index[:, NE // 2:], 16)).reshape(2, NW, NPK)
    pk = jnp.pad(pk, ((0, 0), (0, 0), (0, NPK_IT * 16 - NPK)))
    pk_src, pk_dst = pk[0], pk[1]
    dst_nat = edge_index[1].reshape(NW, NCHUNK, CHUNK)
    zeros64 = jnp.zeros((GROWS, HID), f32)
    zeros16 = jnp.zeros((RPT, 16), f32)
    ones16 = jnp.ones((CHUNK, 16), f32)

    row = lambda v: v.reshape(1, -1)

    counts = _sc_counts(dst_nat, ones16, zeros16)
    h, hw = _embed(x, node_emb_W, row(node_emb_b), W1[0, :HID])

    ea_t = edge_attr.T
    e_ins = (ea_t, ea_t)
    def bd(w):
        a, b = w.shape
        z = jnp.zeros((2 * a, 2 * b), f32)
        return z.at[:a, :b].set(w).at[a:, b:].set(w)

    def b2x(v):
        return jnp.concatenate([v, v]).reshape(1, -1)

    for l in range(NL):
        g = _sc_gather(hw, pk_src)
        g_p = g.reshape(NE // 2, 2 * HID)
        first = l == 0
        want_eout = l < NL - 1
        if want_eout:
            w1ed = bd(W1[l, HID:])
            b1d = b2x(b1[l])
        else:
            w1ed = bd(eW2[l] @ W1[l, HID:])
            b1d = b2x(b1[l] + eb2[l] @ W1[l, HID:])
        outs = _msg(e_ins, g_p, bd(edge_emb_W), b2x(edge_emb_b), bd(eW1[l]),
                    b2x(eb1[l]), bd(eW2[l]), b2x(eb2[l]), w1ed,
                    b1d, bd(W2[l]), b2x(b2[l]), first, want_eout)
        if want_eout:
            m_p, e_next, sums = outs
            e_ins = (e_next,)
        else:
            m_p, sums = outs
        s = _sc_scatter(m_p.reshape(NE, HID), pk_dst, zeros64)
        w1h_next = W1[l + 1, :HID] if want_eout else jnp.zeros((HID, HID), f32)
        ups = _update(s, counts, h, sums, bn_g[l].reshape(1, -1),
                      bn_b[l].reshape(1, -1), w1h_next, want_eout)
        if want_eout:
            h, hw = ups
        else:
            h = ups[0]

    return _pool(h, batch.reshape(NN, 1), rW1, row(rb1), rW2, row(rb2))
